# baseline (speedup 1.0000x reference)
import functools

import jax
import jax.numpy as jnp
from jax import lax
from jax.experimental import pallas as pl
from jax.experimental.pallas import tpu as pltpu
from jax.experimental.pallas import tpu_sc as plsc

MAX_DIST = 20
NUM_HEADS = 16

_NC = 2
_NS = 16
_NW = _NC * _NS
_LANES = 16

_CHUNK = 4096
_GSLICE = 128


def _sc_lookup(dist_hbm, table_hbm, out_hbm, dist_v, idx_v, rows_v, sem):
    b = dist_hbm.shape[0]
    b_per_w = b // _NW
    n_chunks = b_per_w // _CHUNK
    wid = lax.axis_index("s") * _NC + lax.axis_index("c")
    base = wid * b_per_w

    def chunk_body(t, _):
        off = base + t * _CHUNK
        pltpu.sync_copy(dist_hbm.at[pl.ds(off, _CHUNK)], dist_v)

        def clamp_body(j, _):
            v = dist_v[pl.ds(j * _LANES, _LANES)]
            idx_v[pl.ds(j * _LANES, _LANES)] = jnp.clip(v + 1, 0, MAX_DIST + 1)
            return 0

        lax.fori_loop(0, _CHUNK // _LANES, clamp_body, 0, unroll=8)

        copies = []
        for j in range(_CHUNK // _GSLICE):
            copies.append(
                pltpu.make_async_copy(
                    table_hbm.at[idx_v.at[pl.ds(j * _GSLICE, _GSLICE)]],
                    rows_v.at[pl.ds(j * _GSLICE, _GSLICE)],
                    sem,
                )
            )
        for c in copies:
            c.start()
        for c in copies:
            c.wait()

        pltpu.sync_copy(rows_v, out_hbm.at[pl.ds(off, _CHUNK)])
        return 0

    lax.fori_loop(0, n_chunks, chunk_body, 0)


def kernel(dist, table):
    b = dist.size
    flat = dist.reshape((b,)).astype(jnp.int32)
    run = functools.partial(
        pl.kernel,
        out_type=jax.ShapeDtypeStruct((b, NUM_HEADS), jnp.float32),
        mesh=plsc.VectorSubcoreMesh(
            core_axis_name="c", subcore_axis_name="s",
            num_cores=_NC, num_subcores=_NS),
        scratch_types=[
            pltpu.VMEM((_CHUNK,), jnp.int32),
            pltpu.VMEM((_CHUNK,), jnp.int32),
            pltpu.VMEM((_CHUNK, NUM_HEADS), jnp.float32),
            pltpu.SemaphoreType.DMA,
        ],
        compiler_params=pltpu.CompilerParams(use_tc_tiling_on_sc=False),
    )(_sc_lookup)
    out = run(flat, table)
    return out.reshape(dist.shape + (NUM_HEADS,))


# speedup vs baseline: 5.0421x; 5.0421x over previous
import functools

import jax
import jax.numpy as jnp
from jax import lax
from jax.experimental import pallas as pl
from jax.experimental.pallas import tpu as pltpu
from jax.experimental.pallas import tpu_sc as plsc

MAX_DIST = 20
NUM_HEADS = 16

_NC = 2
_NS = 16
_NW = _NC * _NS
_LANES = 16

_CHUNK = 4096
_GSLICE = 128


def _sc_lookup(dist_hbm, table_hbm, out_hbm, dist_v, idx_v, rows_v, tab_v, sem):
    b = dist_hbm.shape[0]
    b_per_w = b // _NW
    n_chunks = b_per_w // _CHUNK
    wid = lax.axis_index("s") * _NC + lax.axis_index("c")
    base = wid * b_per_w

    @pl.when(lax.axis_index("s") == 0)
    def _stage_table():
        pltpu.sync_copy(table_hbm, tab_v)

    plsc.subcore_barrier()

    def chunk_body(t, _):
        off = base + t * _CHUNK
        pltpu.sync_copy(dist_hbm.at[pl.ds(off, _CHUNK)], dist_v)

        def clamp_body(j, _):
            v = dist_v[pl.ds(j * _LANES, _LANES)]
            idx_v[pl.ds(j * _LANES, _LANES)] = jnp.clip(v + 1, 0, MAX_DIST + 1)
            return 0

        lax.fori_loop(0, _CHUNK // _LANES, clamp_body, 0, unroll=8)

        copies = []
        for j in range(_CHUNK // _GSLICE):
            copies.append(
                pltpu.make_async_copy(
                    tab_v.at[idx_v.at[pl.ds(j * _GSLICE, _GSLICE)]],
                    rows_v.at[pl.ds(j * _GSLICE, _GSLICE)],
                    sem,
                )
            )
        for c in copies:
            c.start()
        for c in copies:
            c.wait()

        pltpu.sync_copy(rows_v, out_hbm.at[pl.ds(off, _CHUNK)])
        return 0

    lax.fori_loop(0, n_chunks, chunk_body, 0)


def kernel(dist, table):
    b = dist.size
    flat = dist.reshape((b,)).astype(jnp.int32)
    run = functools.partial(
        pl.kernel,
        out_type=jax.ShapeDtypeStruct((b, NUM_HEADS), jnp.float32),
        mesh=plsc.VectorSubcoreMesh(
            core_axis_name="c", subcore_axis_name="s",
            num_cores=_NC, num_subcores=_NS),
        scratch_types=[
            pltpu.VMEM((_CHUNK,), jnp.int32),
            pltpu.VMEM((_CHUNK,), jnp.int32),
            pltpu.VMEM((_CHUNK, NUM_HEADS), jnp.float32),
            pltpu.VMEM_SHARED((MAX_DIST + 2, NUM_HEADS), jnp.float32),
            pltpu.SemaphoreType.DMA,
        ],
        compiler_params=pltpu.CompilerParams(use_tc_tiling_on_sc=False),
    )(_sc_lookup)
    out = run(flat, table)
    return out.reshape(dist.shape + (NUM_HEADS,))


# speedup vs baseline: 5.1526x; 1.0219x over previous
import functools

import jax
import jax.numpy as jnp
from jax import lax
from jax.experimental import pallas as pl
from jax.experimental.pallas import tpu as pltpu
from jax.experimental.pallas import tpu_sc as plsc

MAX_DIST = 20
NUM_HEADS = 16

_NC = 2
_NS = 16
_NW = _NC * _NS
_LANES = 16

_CHUNK = 2048
_GSLICE = 128
_NBUF = 2


def _sc_lookup(dist_hbm, table_hbm, out_hbm, dist_v, idx_v, rows_v, tab_v,
               isem0, isem1, osem0, osem1, gsem):
    b = dist_hbm.shape[0]
    b_per_w = b // _NW
    n_chunks = b_per_w // _CHUNK
    wid = lax.axis_index("s") * _NC + lax.axis_index("c")
    base = wid * b_per_w
    isems = (isem0, isem1)
    osems = (osem0, osem1)

    @pl.when(lax.axis_index("s") == 0)
    def _stage_table():
        pltpu.sync_copy(table_hbm, tab_v)

    plsc.subcore_barrier()

    def in_copy(t, bi):
        return pltpu.make_async_copy(
            dist_hbm.at[pl.ds(base + t * _CHUNK, _CHUNK)], dist_v.at[bi],
            isems[bi])

    def out_copy(t, bi):
        return pltpu.make_async_copy(
            rows_v.at[bi], out_hbm.at[pl.ds(base + t * _CHUNK, _CHUNK)],
            osems[bi])

    in_copy(0, 0).start()
    in_copy(1, 1).start()

    @pl.loop(0, n_chunks, step=_NBUF)
    def _chunk_pair(t0):
        for bi in range(_NBUF):
            t = t0 + bi
            in_copy(t, bi).wait()

            def clamp_body(j, _):
                v = dist_v[bi, pl.ds(j * _LANES, _LANES)]
                idx_v[bi, pl.ds(j * _LANES, _LANES)] = jnp.clip(
                    v + 1, 0, MAX_DIST + 1)
                return 0

            lax.fori_loop(0, _CHUNK // _LANES, clamp_body, 0, unroll=8)

            @pl.when(t + _NBUF < n_chunks)
            def _prefetch_next():
                in_copy(t + _NBUF, bi).start()

            @pl.when(t >= _NBUF)
            def _drain_prev_writeback():
                out_copy(t - _NBUF, bi).wait()

            copies = []
            for j in range(_CHUNK // _GSLICE):
                copies.append(
                    pltpu.make_async_copy(
                        tab_v.at[idx_v.at[bi].at[pl.ds(j * _GSLICE, _GSLICE)]],
                        rows_v.at[bi].at[pl.ds(j * _GSLICE, _GSLICE)],
                        gsem,
                    )
                )
            for c in copies:
                c.start()
            for c in copies:
                c.wait()

            out_copy(t, bi).start()

    out_copy(n_chunks - 2, 0).wait()
    out_copy(n_chunks - 1, 1).wait()


def kernel(dist, table):
    b = dist.size
    flat = dist.reshape((b,)).astype(jnp.int32)
    run = functools.partial(
        pl.kernel,
        out_type=jax.ShapeDtypeStruct((b, NUM_HEADS), jnp.float32),
        mesh=plsc.VectorSubcoreMesh(
            core_axis_name="c", subcore_axis_name="s",
            num_cores=_NC, num_subcores=_NS),
        scratch_types=[
            pltpu.VMEM((_NBUF, _CHUNK), jnp.int32),
            pltpu.VMEM((_NBUF, _CHUNK), jnp.int32),
            pltpu.VMEM((_NBUF, _CHUNK, NUM_HEADS), jnp.float32),
            pltpu.VMEM_SHARED((MAX_DIST + 2, NUM_HEADS), jnp.float32),
            pltpu.SemaphoreType.DMA,
            pltpu.SemaphoreType.DMA,
            pltpu.SemaphoreType.DMA,
            pltpu.SemaphoreType.DMA,
            pltpu.SemaphoreType.DMA,
        ],
        compiler_params=pltpu.CompilerParams(use_tc_tiling_on_sc=False),
    )(_sc_lookup)
    out = run(flat, table)
    return out.reshape(dist.shape + (NUM_HEADS,))
